# baseline (speedup 1.0000x reference)
import functools

import jax
import jax.numpy as jnp
from jax import lax
from jax.experimental import pallas as pl
from jax.experimental.pallas import tpu as pltpu
from jax.experimental.pallas import tpu_sc as plsc

_EPS = 1e-5
_N_EDGES = 20000
_NC = 2
_NS = 16
_NW = _NC * _NS
_K = 80
_ZCH = 32


def _sc_segsum(tables, gidx, sidx, num_segments, k=80, with_count=False,
               count_segments=0):
    nnz = gidx.shape[0]
    w = tables[0].shape[1]
    assert all(t.shape[1] == w for t in tables)
    nt = len(tables)
    assert w % 16 == 0
    npad = -(-num_segments // (_NS * _ZCH)) * (_NS * _ZCH)
    rpt = npad // _NS
    npad_c = -(-count_segments // (_NS * _ZCH)) * (_NS * _ZCH)
    rpt_c = npad_c // _NS

    nnz_pad = -(-nnz // (_NW * k)) * (_NW * k)
    if nnz_pad != nnz:
        assert not with_count and npad > num_segments
        ext = nnz_pad - nnz
        gidx = jnp.concatenate([gidx, jnp.zeros((ext,), jnp.int32)])
        sidx = jnp.concatenate(
            [sidx, jnp.full((ext,), npad - 1, jnp.int32)])
    q = nnz_pad // _NW
    nbt = q // k

    gidx2 = gidx.reshape(-1, k)
    sidx2 = sidx.reshape(-1, k)

    nbuf = 3 if k > 80 else 4

    out_type = [jax.ShapeDtypeStruct((_NC, npad, w), jnp.float32)
                for _ in range(nt)]
    scratch = [
        pltpu.VMEM_SHARED((npad, w), jnp.float32),
        pltpu.VMEM((nbt, k), jnp.int32),
        pltpu.VMEM((nbt, k), jnp.int32),
    ] + [pltpu.VMEM((k, w), jnp.float32) for _ in range(nbuf)] + [
        pltpu.VMEM((_ZCH, w), jnp.float32),
    ] + [pltpu.SemaphoreType.DMA for _ in range(nbuf)]
    if with_count:
        out_type.append(
            jax.ShapeDtypeStruct((_NC, npad_c, 16), jnp.float32))
        scratch += [
            pltpu.VMEM_SHARED((npad_c, 16), jnp.float32),
            pltpu.VMEM((_ZCH, 16), jnp.float32),
            pltpu.VMEM((k, 16), jnp.float32),
        ]

    def body(*args):
        tables_h = args[:nt]
        gidx_h, sidx_h = args[nt], args[nt + 1]
        outs_h = args[nt + 2:2 * nt + 2]
        rest = args[2 * nt + 2:]
        if with_count:
            cnt_h = rest[0]
            rest = rest[1:]
        acc_sh, gi_all, si_all = rest[0], rest[1], rest[2]
        rows = rest[3:3 + nbuf]
        zbuf = rest[3 + nbuf]
        sems = rest[4 + nbuf:4 + 2 * nbuf]
        if with_count:
            cnt_sh, zbuf_c, ones_v = rest[4 + 2 * nbuf:]
        cid = lax.axis_index("c")
        sid = lax.axis_index("s")
        zv = jnp.zeros((16,), jnp.float32)
        wid = cid * _NS + sid

        pltpu.sync_copy(gidx_h.at[pl.ds(wid * nbt, nbt)], gi_all)
        pltpu.sync_copy(sidx_h.at[pl.ds(wid * nbt, nbt)], si_all)

        def fire(tbl, b, j):
            pltpu.async_copy(tbl.at[gi_all.at[b]], rows[j], sems[j])

        def prologue(tbl):
            for j in range(nbuf - 1):
                fire(tbl, j, j)

        prologue(tables_h[0])

        def zero_acc():
            @pl.loop(0, rpt // _ZCH)
            def _(i):
                pltpu.sync_copy(
                    zbuf, acc_sh.at[pl.ds(sid * rpt + i * _ZCH, _ZCH)])

        @pl.loop(0, _ZCH)
        def _(r):
            for c in range(w // 16):
                zbuf[r, pl.ds(c * 16, 16)] = zv

        zero_acc()

        if with_count:
            onev = jnp.where(lax.iota(jnp.int32, 16) == 0, 1.0, 0.0)

            @pl.loop(0, _ZCH)
            def _(r):
                zbuf_c[r, :] = zv

            @pl.loop(0, rpt_c // _ZCH)
            def _(i):
                pltpu.sync_copy(
                    zbuf_c, cnt_sh.at[pl.ds(sid * rpt_c + i * _ZCH, _ZCH)])

            @pl.loop(0, k)
            def _(r):
                ones_v[r, :] = onev

        def wait_gather(j):
            pltpu.make_async_copy(
                tables_h[0].at[gi_all.at[0]], rows[j], sems[j]).wait()

        def retire(b, j):
            wait_gather(j)
            pltpu.sync_copy(rows[j], acc_sh.at[si_all.at[b]], add=True)
            if with_count:
                pltpu.sync_copy(ones_v, cnt_sh.at[gi_all.at[b]], add=True)

        n_main = (nbt - (nbuf - 1)) // nbuf

        for t, tbl in enumerate(tables_h):
            plsc.subcore_barrier()

            @pl.loop(0, n_main)
            def _(i):
                b0 = nbuf * i
                for j in range(nbuf):
                    fire(tbl, b0 + j + nbuf - 1, (j + nbuf - 1) % nbuf)
                    retire(b0 + j, j)

            base = n_main * nbuf
            for b in range(base, nbt):
                m = b + nbuf - 1
                if m < nbt:
                    fire(tbl, m, m % nbuf)
                retire(b, b % nbuf)

            if t + 1 < nt:
                prologue(tables_h[t + 1])

            plsc.subcore_barrier()

            r0 = sid * rpt
            pltpu.sync_copy(acc_sh.at[pl.ds(r0, rpt)],
                            outs_h[t].at[cid, pl.ds(r0, rpt)])
            if t + 1 < nt:
                zero_acc()

        if with_count:
            rc = sid * rpt_c
            pltpu.sync_copy(cnt_sh.at[pl.ds(rc, rpt_c)],
                            cnt_h.at[cid, pl.ds(rc, rpt_c)])

    mesh = plsc.VectorSubcoreMesh(core_axis_name="c", subcore_axis_name="s")
    f = pl.kernel(body, out_type=out_type, mesh=mesh, scratch_types=scratch,
                  compiler_params=pltpu.CompilerParams(
                      use_tc_tiling_on_sc=False),
                  name=f"sc_segsum_w{w}x{nt}{'_cnt' if with_count else ''}")
    return f(*tables, gidx2, sidx2)


def _prep_body(x_ref, x0_ref, x1_ref, y0_ref, y1_ref, s_ref):
    x = x_ref[...]
    mu = jnp.mean(x, axis=1, keepdims=True)
    var = jnp.mean(jnp.square(x - mu), axis=1, keepdims=True)
    inv = lax.rsqrt(var + _EPS)
    y = x * inv
    x0_ref[...] = x[:, :64]
    x1_ref[...] = x[:, 64:]
    y0_ref[...] = y[:, :64]
    y1_ref[...] = y[:, 64:]
    t = mu * inv
    r = x.shape[0]
    col = lax.broadcasted_iota(jnp.int32, (r, 16), 1)
    s_ref[...] = jnp.where(col == 0, t, jnp.where(col == 1, 1.0, 0.0))


def _mid_body(x0a, x0b, x1a, x1b, y0a, y0b, y1a, y1b, sa, sb,
              ig, ib, og, ob, e_ref, em_ref):
    sx = jnp.concatenate([x0a[0] + x0b[0], x1a[0] + x1b[0]], axis=1)
    sy = jnp.concatenate([y0a[0] + y0b[0], y1a[0] + y1b[0]], axis=1)
    s = sa[0] + sb[0]
    t_s = s[:, 0:1]
    deg = s[:, 1:2]
    sxn = sy * ig[...] - t_s * ig[...] + deg * ib[...]
    pos = deg > 0.0
    agg = jnp.where(pos, sxn / jnp.where(pos, deg * deg, 1.0), 0.0)
    mu = jnp.mean(agg, axis=1, keepdims=True)
    var = jnp.mean(jnp.square(agg - mu), axis=1, keepdims=True)
    e_ref[...] = (agg - mu) * lax.rsqrt(var + _EPS) * og[...] + ob[...]
    em_ref[...] = sx / jnp.maximum(deg, 1.0)


def _fin_body(na, nb_, ca, cb, w_ref, b_ref, o_ref):
    nsum = na[0] + nb_[0]
    deg = (ca[0] + cb[0])[:, 0:1]
    nmean = nsum / jnp.maximum(deg, 1.0)
    o_ref[...] = jnp.dot(nmean, w_ref[...],
                         preferred_element_type=jnp.float32) + b_ref[...]


@jax.jit
def kernel(x, edge_index, in_gamma, in_beta, out_gamma, out_beta,
           W_enc, b_enc):
    src = edge_index[0]
    dst = edge_index[1]
    n, d = x.shape
    ne = _N_EDGES
    rb = 1000

    x0, x1, y0, y1, s16 = pl.pallas_call(
        _prep_body,
        grid=(n // rb,),
        in_specs=[pl.BlockSpec((rb, d), lambda i: (i, 0))],
        out_specs=[
            pl.BlockSpec((rb, 64), lambda i: (i, 0)),
            pl.BlockSpec((rb, 64), lambda i: (i, 0)),
            pl.BlockSpec((rb, 64), lambda i: (i, 0)),
            pl.BlockSpec((rb, 64), lambda i: (i, 0)),
            pl.BlockSpec((rb, 16), lambda i: (i, 0)),
        ],
        out_shape=[
            jax.ShapeDtypeStruct((n, 64), jnp.float32),
            jax.ShapeDtypeStruct((n, 64), jnp.float32),
            jax.ShapeDtypeStruct((n, 64), jnp.float32),
            jax.ShapeDtypeStruct((n, 64), jnp.float32),
            jax.ShapeDtypeStruct((n, 16), jnp.float32),
        ],
    )(x)

    accx0, accx1, accy0, accy1 = _sc_segsum([x0, x1, y0, y1], src, dst, ne,
                                            k=128)
    accs, cntn = _sc_segsum([s16], src, dst, ne, with_count=True,
                            count_segments=n)

    nb_e = ne // rb
    ig = in_gamma.reshape(1, d)
    ib = in_beta.reshape(1, d)
    og = out_gamma.reshape(1, d)
    ob = out_beta.reshape(1, d)
    spec64a = pl.BlockSpec((1, rb, 64), lambda i: (0, i, 0))
    spec64b = pl.BlockSpec((1, rb, 64), lambda i: (1, i, 0))
    spec16a = pl.BlockSpec((1, rb, 16), lambda i: (0, i, 0))
    spec16b = pl.BlockSpec((1, rb, 16), lambda i: (1, i, 0))
    vspec = pl.BlockSpec((1, d), lambda i: (0, 0))
    e, emean = pl.pallas_call(
        _mid_body,
        grid=(nb_e,),
        in_specs=[spec64a, spec64b, spec64a, spec64b,
                  spec64a, spec64b, spec64a, spec64b, spec16a, spec16b,
                  vspec, vspec, vspec, vspec],
        out_specs=[pl.BlockSpec((rb, d), lambda i: (i, 0)),
                   pl.BlockSpec((rb, d), lambda i: (i, 0))],
        out_shape=[jax.ShapeDtypeStruct((ne, d), jnp.float32),
                   jax.ShapeDtypeStruct((ne, d), jnp.float32)],
    )(accx0, accx0, accx1, accx1,
      accy0, accy0, accy1, accy1,
      accs, accs, ig, ib, og, ob)

    accn, = _sc_segsum([emean], dst, src, n, k=40)

    nb_n = n // rb
    specda = pl.BlockSpec((1, rb, d), lambda i: (0, i, 0))
    specdb = pl.BlockSpec((1, rb, d), lambda i: (1, i, 0))
    x_out = pl.pallas_call(
        _fin_body,
        grid=(nb_n,),
        in_specs=[specda, specdb, spec16a, spec16b,
                  pl.BlockSpec((d, d), lambda i: (0, 0)), vspec],
        out_specs=pl.BlockSpec((rb, d), lambda i: (i, 0)),
        out_shape=jax.ShapeDtypeStruct((n, d), jnp.float32),
    )(accn, accn, cntn, cntn,
      W_enc, b_enc.reshape(1, d))

    return (x_out, e)


# speedup vs baseline: 1.6317x; 1.6317x over previous
import functools

import jax
import jax.numpy as jnp
from jax import lax
from jax.experimental import pallas as pl
from jax.experimental.pallas import tpu as pltpu
from jax.experimental.pallas import tpu_sc as plsc

_EPS = 1e-5
_N_EDGES = 20000
_NC = 2
_NS = 16
_NW = _NC * _NS
_K = 80
_ZCH = 32


def _sc_segsum(tables, gidx, sidx, num_segments, k=80, with_count=False,
               count_segments=0):
    nnz = gidx.shape[0]
    w = tables[0].shape[1]
    assert all(t.shape[1] == w for t in tables)
    nt = len(tables)
    assert w % 16 == 0
    npad = -(-num_segments // (_NS * _ZCH)) * (_NS * _ZCH)
    rpt = npad // _NS
    npad_c = -(-count_segments // (_NS * _ZCH)) * (_NS * _ZCH)
    rpt_c = npad_c // _NS

    nnz_pad = -(-nnz // (_NW * k)) * (_NW * k)
    if nnz_pad != nnz:
        assert not with_count and npad > num_segments
        ext = nnz_pad - nnz
        gidx = jnp.concatenate([gidx, jnp.zeros((ext,), jnp.int32)])
        sidx = jnp.concatenate(
            [sidx, jnp.full((ext,), npad - 1, jnp.int32)])
    q = nnz_pad // _NW
    nbt = q // k

    gidx2 = gidx.reshape(-1, k)
    sidx2 = sidx.reshape(-1, k)

    nbuf = 3 if k > 80 else 4

    out_type = [jax.ShapeDtypeStruct((_NC, npad, w), jnp.float32)
                for _ in range(nt)]
    scratch = [
        pltpu.VMEM_SHARED((npad, w), jnp.float32),
        pltpu.VMEM((nbt, k), jnp.int32),
        pltpu.VMEM((nbt, k), jnp.int32),
    ] + [pltpu.VMEM((k, w), jnp.float32) for _ in range(nbuf)] + [
        pltpu.VMEM((_ZCH, w), jnp.float32),
    ] + [pltpu.SemaphoreType.DMA for _ in range(nbuf)]
    if with_count:
        out_type.append(
            jax.ShapeDtypeStruct((_NC, npad_c, 16), jnp.float32))
        scratch += [
            pltpu.VMEM_SHARED((npad_c, 16), jnp.float32),
            pltpu.VMEM((_ZCH, 16), jnp.float32),
            pltpu.VMEM((k, 16), jnp.float32),
        ]

    def body(*args):
        tables_h = args[:nt]
        gidx_h, sidx_h = args[nt], args[nt + 1]
        outs_h = args[nt + 2:2 * nt + 2]
        rest = args[2 * nt + 2:]
        if with_count:
            cnt_h = rest[0]
            rest = rest[1:]
        acc_sh, gi_all, si_all = rest[0], rest[1], rest[2]
        rows = rest[3:3 + nbuf]
        zbuf = rest[3 + nbuf]
        sems = rest[4 + nbuf:4 + 2 * nbuf]
        if with_count:
            cnt_sh, zbuf_c, ones_v = rest[4 + 2 * nbuf:]
        cid = lax.axis_index("c")
        sid = lax.axis_index("s")
        zv = jnp.zeros((16,), jnp.float32)
        wid = cid * _NS + sid

        pltpu.sync_copy(gidx_h.at[pl.ds(wid * nbt, nbt)], gi_all)
        pltpu.sync_copy(sidx_h.at[pl.ds(wid * nbt, nbt)], si_all)

        def fire(tbl, b, j):
            pltpu.async_copy(tbl.at[gi_all.at[b]], rows[j], sems[j])

        def prologue(tbl):
            for j in range(nbuf - 1):
                fire(tbl, j, j)

        prologue(tables_h[0])

        def zero_acc():
            @pl.loop(0, rpt // _ZCH)
            def _(i):
                pltpu.sync_copy(
                    zbuf, acc_sh.at[pl.ds(sid * rpt + i * _ZCH, _ZCH)])

        @pl.loop(0, _ZCH)
        def _(r):
            for c in range(w // 16):
                zbuf[r, pl.ds(c * 16, 16)] = zv

        zero_acc()

        if with_count:
            onev = jnp.where(lax.iota(jnp.int32, 16) == 0, 1.0, 0.0)

            @pl.loop(0, _ZCH)
            def _(r):
                zbuf_c[r, :] = zv

            @pl.loop(0, rpt_c // _ZCH)
            def _(i):
                pltpu.sync_copy(
                    zbuf_c, cnt_sh.at[pl.ds(sid * rpt_c + i * _ZCH, _ZCH)])

            @pl.loop(0, k)
            def _(r):
                ones_v[r, :] = onev

        def wait_gather(j):
            pltpu.make_async_copy(
                tables_h[0].at[gi_all.at[0]], rows[j], sems[j]).wait()

        def retire(b, j):
            wait_gather(j)
            pltpu.sync_copy(rows[j], acc_sh.at[si_all.at[b]], add=True)
            if with_count:
                pltpu.sync_copy(ones_v, cnt_sh.at[gi_all.at[b]], add=True)

        n_main = (nbt - (nbuf - 1)) // nbuf

        for t, tbl in enumerate(tables_h):
            plsc.subcore_barrier()

            @pl.loop(0, n_main)
            def _(i):
                b0 = nbuf * i
                for j in range(nbuf):
                    fire(tbl, b0 + j + nbuf - 1, (j + nbuf - 1) % nbuf)
                    retire(b0 + j, j)

            base = n_main * nbuf
            for b in range(base, nbt):
                m = b + nbuf - 1
                if m < nbt:
                    fire(tbl, m, m % nbuf)
                retire(b, b % nbuf)

            if t + 1 < nt:
                prologue(tables_h[t + 1])

            plsc.subcore_barrier()

            r0 = sid * rpt
            pltpu.sync_copy(acc_sh.at[pl.ds(r0, rpt)],
                            outs_h[t].at[cid, pl.ds(r0, rpt)])
            if t + 1 < nt:
                zero_acc()

        if with_count:
            rc = sid * rpt_c
            pltpu.sync_copy(cnt_sh.at[pl.ds(rc, rpt_c)],
                            cnt_h.at[cid, pl.ds(rc, rpt_c)])

    mesh = plsc.VectorSubcoreMesh(core_axis_name="c", subcore_axis_name="s")
    f = pl.kernel(body, out_type=out_type, mesh=mesh, scratch_types=scratch,
                  compiler_params=pltpu.CompilerParams(
                      use_tc_tiling_on_sc=False),
                  name=f"sc_segsum_w{w}x{nt}{'_cnt' if with_count else ''}")
    return f(*tables, gidx2, sidx2)


def _prep_body(x_ref, x0_ref, x1_ref, y0_ref, y1_ref, s_ref):
    x = x_ref[...]
    mu = jnp.mean(x, axis=1, keepdims=True)
    var = jnp.mean(jnp.square(x - mu), axis=1, keepdims=True)
    inv = lax.rsqrt(var + _EPS)
    y = x * inv
    x0_ref[...] = x[:, :64]
    x1_ref[...] = x[:, 64:]
    y0_ref[...] = y[:, :64]
    y1_ref[...] = y[:, 64:]
    t = mu * inv
    r = x.shape[0]
    col = lax.broadcasted_iota(jnp.int32, (r, 16), 1)
    s_ref[...] = jnp.where(col == 0, t, jnp.where(col == 1, 1.0, 0.0))


def _mid_body(x0a, x0b, x1a, x1b, y0a, y0b, y1a, y1b, sa, sb,
              ig, ib, og, ob, e_ref, em_ref):
    sx = jnp.concatenate([x0a[0] + x0b[0], x1a[0] + x1b[0]], axis=1)
    sy = jnp.concatenate([y0a[0] + y0b[0], y1a[0] + y1b[0]], axis=1)
    s = sa[0] + sb[0]
    t_s = s[:, 0:1]
    deg = s[:, 1:2]
    sxn = sy * ig[...] - t_s * ig[...] + deg * ib[...]
    pos = deg > 0.0
    agg = jnp.where(pos, sxn / jnp.where(pos, deg * deg, 1.0), 0.0)
    mu = jnp.mean(agg, axis=1, keepdims=True)
    var = jnp.mean(jnp.square(agg - mu), axis=1, keepdims=True)
    e_ref[...] = (agg - mu) * lax.rsqrt(var + _EPS) * og[...] + ob[...]
    em_ref[...] = sx / jnp.maximum(deg, 1.0)


def _fin_body(na, nb_, ca, cb, w_ref, b_ref, o_ref):
    nsum = na[0] + nb_[0]
    deg = (ca[0] + cb[0])[:, 0:1]
    nmean = nsum / jnp.maximum(deg, 1.0)
    o_ref[...] = jnp.dot(nmean, w_ref[...],
                         preferred_element_type=jnp.float32) + b_ref[...]


@jax.jit
def kernel(x, edge_index, in_gamma, in_beta, out_gamma, out_beta,
           W_enc, b_enc):
    src = edge_index[0]
    dst = edge_index[1]
    n, d = x.shape
    ne = _N_EDGES
    rb = 1000

    x0, x1, y0, y1, s16 = pl.pallas_call(
        _prep_body,
        grid=(n // rb,),
        in_specs=[pl.BlockSpec((rb, d), lambda i: (i, 0))],
        out_specs=[
            pl.BlockSpec((rb, 64), lambda i: (i, 0)),
            pl.BlockSpec((rb, 64), lambda i: (i, 0)),
            pl.BlockSpec((rb, 64), lambda i: (i, 0)),
            pl.BlockSpec((rb, 64), lambda i: (i, 0)),
            pl.BlockSpec((rb, 16), lambda i: (i, 0)),
        ],
        out_shape=[
            jax.ShapeDtypeStruct((n, 64), jnp.float32),
            jax.ShapeDtypeStruct((n, 64), jnp.float32),
            jax.ShapeDtypeStruct((n, 64), jnp.float32),
            jax.ShapeDtypeStruct((n, 64), jnp.float32),
            jax.ShapeDtypeStruct((n, 16), jnp.float32),
        ],
    )(x)

    accx0, accx1, accy0, accy1 = _sc_segsum([x0, x1, y0, y1], src, dst, ne)
    accs, cntn = _sc_segsum([s16], src, dst, ne, with_count=True,
                            count_segments=n)

    nb_e = ne // rb
    ig = in_gamma.reshape(1, d)
    ib = in_beta.reshape(1, d)
    og = out_gamma.reshape(1, d)
    ob = out_beta.reshape(1, d)
    spec64a = pl.BlockSpec((1, rb, 64), lambda i: (0, i, 0))
    spec64b = pl.BlockSpec((1, rb, 64), lambda i: (1, i, 0))
    spec16a = pl.BlockSpec((1, rb, 16), lambda i: (0, i, 0))
    spec16b = pl.BlockSpec((1, rb, 16), lambda i: (1, i, 0))
    vspec = pl.BlockSpec((1, d), lambda i: (0, 0))
    e, emean = pl.pallas_call(
        _mid_body,
        grid=(nb_e,),
        in_specs=[spec64a, spec64b, spec64a, spec64b,
                  spec64a, spec64b, spec64a, spec64b, spec16a, spec16b,
                  vspec, vspec, vspec, vspec],
        out_specs=[pl.BlockSpec((rb, d), lambda i: (i, 0)),
                   pl.BlockSpec((rb, d), lambda i: (i, 0))],
        out_shape=[jax.ShapeDtypeStruct((ne, d), jnp.float32),
                   jax.ShapeDtypeStruct((ne, d), jnp.float32)],
    )(accx0, accx0, accx1, accx1,
      accy0, accy0, accy1, accy1,
      accs, accs, ig, ib, og, ob)

    accn, = _sc_segsum([emean], dst, src, n, k=40)

    nb_n = n // rb
    specda = pl.BlockSpec((1, rb, d), lambda i: (0, i, 0))
    specdb = pl.BlockSpec((1, rb, d), lambda i: (1, i, 0))
    x_out = pl.pallas_call(
        _fin_body,
        grid=(nb_n,),
        in_specs=[specda, specdb, spec16a, spec16b,
                  pl.BlockSpec((d, d), lambda i: (0, 0)), vspec],
        out_specs=pl.BlockSpec((rb, d), lambda i: (i, 0)),
        out_shape=jax.ShapeDtypeStruct((n, d), jnp.float32),
    )(accn, accn, cntn, cntn,
      W_enc, b_enc.reshape(1, d))

    return (x_out, e)


# speedup vs baseline: 1.6408x; 1.0056x over previous
import functools

import jax
import jax.numpy as jnp
from jax import lax
from jax.experimental import pallas as pl
from jax.experimental.pallas import tpu as pltpu
from jax.experimental.pallas import tpu_sc as plsc

_EPS = 1e-5
_N_EDGES = 20000
_NC = 2
_NS = 16
_NW = _NC * _NS
_K = 80
_ZCH = 16


def _sc_segsum(tables, gidx, sidx, num_segments, k=80, nbuf=4,
               with_count=False, count_segments=0):
    nnz = gidx.shape[0]
    w = tables[0].shape[1]
    assert all(t.shape[1] == w for t in tables)
    nt = len(tables)
    assert w % 16 == 0
    npad = -(-num_segments // (_NS * _ZCH)) * (_NS * _ZCH)
    rpt = npad // _NS
    npad_c = -(-count_segments // (_NS * _ZCH)) * (_NS * _ZCH)
    rpt_c = npad_c // _NS

    nnz_pad = -(-nnz // (_NW * k)) * (_NW * k)
    if nnz_pad != nnz:
        assert not with_count and npad > num_segments
        ext = nnz_pad - nnz
        gidx = jnp.concatenate([gidx, jnp.zeros((ext,), jnp.int32)])
        sidx = jnp.concatenate(
            [sidx, jnp.full((ext,), npad - 1, jnp.int32)])
    q = nnz_pad // _NW
    nbt = q // k

    gidx2 = gidx.reshape(-1, k)
    sidx2 = sidx.reshape(-1, k)


    out_type = [jax.ShapeDtypeStruct((_NC, npad, w), jnp.float32)
                for _ in range(nt)]
    scratch = [
        pltpu.VMEM_SHARED((npad, w), jnp.float32),
        pltpu.VMEM((nbt, k), jnp.int32),
        pltpu.VMEM((nbt, k), jnp.int32),
    ] + [pltpu.VMEM((k, w), jnp.float32) for _ in range(nbuf)] + [
        pltpu.VMEM((_ZCH, w), jnp.float32),
    ] + [pltpu.SemaphoreType.DMA for _ in range(nbuf)]
    if with_count:
        out_type.append(
            jax.ShapeDtypeStruct((_NC, npad_c, 16), jnp.float32))
        scratch += [
            pltpu.VMEM_SHARED((npad_c, 16), jnp.float32),
            pltpu.VMEM((_ZCH, 16), jnp.float32),
            pltpu.VMEM((k, 16), jnp.float32),
        ]

    def body(*args):
        tables_h = args[:nt]
        gidx_h, sidx_h = args[nt], args[nt + 1]
        outs_h = args[nt + 2:2 * nt + 2]
        rest = args[2 * nt + 2:]
        if with_count:
            cnt_h = rest[0]
            rest = rest[1:]
        acc_sh, gi_all, si_all = rest[0], rest[1], rest[2]
        rows = rest[3:3 + nbuf]
        zbuf = rest[3 + nbuf]
        sems = rest[4 + nbuf:4 + 2 * nbuf]
        if with_count:
            cnt_sh, zbuf_c, ones_v = rest[4 + 2 * nbuf:]
        cid = lax.axis_index("c")
        sid = lax.axis_index("s")
        zv = jnp.zeros((16,), jnp.float32)
        wid = cid * _NS + sid

        pltpu.sync_copy(gidx_h.at[pl.ds(wid * nbt, nbt)], gi_all)
        pltpu.sync_copy(sidx_h.at[pl.ds(wid * nbt, nbt)], si_all)

        def fire(tbl, b, j):
            pltpu.async_copy(tbl.at[gi_all.at[b]], rows[j], sems[j])

        def prologue(tbl):
            for j in range(nbuf - 1):
                fire(tbl, j, j)

        prologue(tables_h[0])

        def zero_acc():
            @pl.loop(0, rpt // _ZCH)
            def _(i):
                pltpu.sync_copy(
                    zbuf, acc_sh.at[pl.ds(sid * rpt + i * _ZCH, _ZCH)])

        @pl.loop(0, _ZCH)
        def _(r):
            for c in range(w // 16):
                zbuf[r, pl.ds(c * 16, 16)] = zv

        zero_acc()

        if with_count:
            onev = jnp.where(lax.iota(jnp.int32, 16) == 0, 1.0, 0.0)

            @pl.loop(0, _ZCH)
            def _(r):
                zbuf_c[r, :] = zv

            @pl.loop(0, rpt_c // _ZCH)
            def _(i):
                pltpu.sync_copy(
                    zbuf_c, cnt_sh.at[pl.ds(sid * rpt_c + i * _ZCH, _ZCH)])

            @pl.loop(0, k)
            def _(r):
                ones_v[r, :] = onev

        def wait_gather(j):
            pltpu.make_async_copy(
                tables_h[0].at[gi_all.at[0]], rows[j], sems[j]).wait()

        def retire(b, j):
            wait_gather(j)
            pltpu.sync_copy(rows[j], acc_sh.at[si_all.at[b]], add=True)
            if with_count:
                pltpu.sync_copy(ones_v, cnt_sh.at[gi_all.at[b]], add=True)

        n_main = (nbt - (nbuf - 1)) // nbuf

        for t, tbl in enumerate(tables_h):
            plsc.subcore_barrier()

            @pl.loop(0, n_main)
            def _(i):
                b0 = nbuf * i
                for j in range(nbuf):
                    fire(tbl, b0 + j + nbuf - 1, (j + nbuf - 1) % nbuf)
                    retire(b0 + j, j)

            base = n_main * nbuf
            for b in range(base, nbt):
                m = b + nbuf - 1
                if m < nbt:
                    fire(tbl, m, m % nbuf)
                retire(b, b % nbuf)

            if t + 1 < nt:
                prologue(tables_h[t + 1])

            plsc.subcore_barrier()

            r0 = sid * rpt
            pltpu.sync_copy(acc_sh.at[pl.ds(r0, rpt)],
                            outs_h[t].at[cid, pl.ds(r0, rpt)])
            if t + 1 < nt:
                zero_acc()

        if with_count:
            rc = sid * rpt_c
            pltpu.sync_copy(cnt_sh.at[pl.ds(rc, rpt_c)],
                            cnt_h.at[cid, pl.ds(rc, rpt_c)])

    mesh = plsc.VectorSubcoreMesh(core_axis_name="c", subcore_axis_name="s")
    f = pl.kernel(body, out_type=out_type, mesh=mesh, scratch_types=scratch,
                  compiler_params=pltpu.CompilerParams(
                      use_tc_tiling_on_sc=False),
                  name=f"sc_segsum_w{w}x{nt}{'_cnt' if with_count else ''}")
    return f(*tables, gidx2, sidx2)


def _prep_body(x_ref, x0_ref, x1_ref, y0_ref, y1_ref, s_ref):
    x = x_ref[...]
    mu = jnp.mean(x, axis=1, keepdims=True)
    var = jnp.mean(jnp.square(x - mu), axis=1, keepdims=True)
    inv = lax.rsqrt(var + _EPS)
    y = x * inv
    x0_ref[...] = x[:, :64]
    x1_ref[...] = x[:, 64:]
    y0_ref[...] = y[:, :64]
    y1_ref[...] = y[:, 64:]
    t = mu * inv
    r = x.shape[0]
    col = lax.broadcasted_iota(jnp.int32, (r, 16), 1)
    s_ref[...] = jnp.where(col == 0, t, jnp.where(col == 1, 1.0, 0.0))


def _mid_body(x0a, x0b, x1a, x1b, y0a, y0b, y1a, y1b, sa, sb,
              ig, ib, og, ob, e_ref, em_ref):
    sx = jnp.concatenate([x0a[0] + x0b[0], x1a[0] + x1b[0]], axis=1)
    sy = jnp.concatenate([y0a[0] + y0b[0], y1a[0] + y1b[0]], axis=1)
    s = sa[0] + sb[0]
    t_s = s[:, 0:1]
    deg = s[:, 1:2]
    sxn = sy * ig[...] - t_s * ig[...] + deg * ib[...]
    pos = deg > 0.0
    agg = jnp.where(pos, sxn / jnp.where(pos, deg * deg, 1.0), 0.0)
    mu = jnp.mean(agg, axis=1, keepdims=True)
    var = jnp.mean(jnp.square(agg - mu), axis=1, keepdims=True)
    e_ref[...] = (agg - mu) * lax.rsqrt(var + _EPS) * og[...] + ob[...]
    em_ref[...] = sx / jnp.maximum(deg, 1.0)


def _fin_body(na, nb_, ca, cb, w_ref, b_ref, o_ref):
    nsum = na[0] + nb_[0]
    deg = (ca[0] + cb[0])[:, 0:1]
    nmean = nsum / jnp.maximum(deg, 1.0)
    o_ref[...] = jnp.dot(nmean, w_ref[...],
                         preferred_element_type=jnp.float32) + b_ref[...]


@jax.jit
def kernel(x, edge_index, in_gamma, in_beta, out_gamma, out_beta,
           W_enc, b_enc):
    src = edge_index[0]
    dst = edge_index[1]
    n, d = x.shape
    ne = _N_EDGES
    rb = 1000

    x0, x1, y0, y1, s16 = pl.pallas_call(
        _prep_body,
        grid=(n // rb,),
        in_specs=[pl.BlockSpec((rb, d), lambda i: (i, 0))],
        out_specs=[
            pl.BlockSpec((rb, 64), lambda i: (i, 0)),
            pl.BlockSpec((rb, 64), lambda i: (i, 0)),
            pl.BlockSpec((rb, 64), lambda i: (i, 0)),
            pl.BlockSpec((rb, 64), lambda i: (i, 0)),
            pl.BlockSpec((rb, 16), lambda i: (i, 0)),
        ],
        out_shape=[
            jax.ShapeDtypeStruct((n, 64), jnp.float32),
            jax.ShapeDtypeStruct((n, 64), jnp.float32),
            jax.ShapeDtypeStruct((n, 64), jnp.float32),
            jax.ShapeDtypeStruct((n, 64), jnp.float32),
            jax.ShapeDtypeStruct((n, 16), jnp.float32),
        ],
    )(x)

    accx0, accx1, accy0, accy1 = _sc_segsum([x0, x1, y0, y1], src, dst, ne,
                                            nbuf=5)
    accs, cntn = _sc_segsum([s16], src, dst, ne, nbuf=8, with_count=True,
                            count_segments=n)

    nb_e = ne // rb
    ig = in_gamma.reshape(1, d)
    ib = in_beta.reshape(1, d)
    og = out_gamma.reshape(1, d)
    ob = out_beta.reshape(1, d)
    spec64a = pl.BlockSpec((1, rb, 64), lambda i: (0, i, 0))
    spec64b = pl.BlockSpec((1, rb, 64), lambda i: (1, i, 0))
    spec16a = pl.BlockSpec((1, rb, 16), lambda i: (0, i, 0))
    spec16b = pl.BlockSpec((1, rb, 16), lambda i: (1, i, 0))
    vspec = pl.BlockSpec((1, d), lambda i: (0, 0))
    e, emean = pl.pallas_call(
        _mid_body,
        grid=(nb_e,),
        in_specs=[spec64a, spec64b, spec64a, spec64b,
                  spec64a, spec64b, spec64a, spec64b, spec16a, spec16b,
                  vspec, vspec, vspec, vspec],
        out_specs=[pl.BlockSpec((rb, d), lambda i: (i, 0)),
                   pl.BlockSpec((rb, d), lambda i: (i, 0))],
        out_shape=[jax.ShapeDtypeStruct((ne, d), jnp.float32),
                   jax.ShapeDtypeStruct((ne, d), jnp.float32)],
    )(accx0, accx0, accx1, accx1,
      accy0, accy0, accy1, accy1,
      accs, accs, ig, ib, og, ob)

    accn, = _sc_segsum([emean], dst, src, n, k=40, nbuf=5)

    nb_n = n // rb
    specda = pl.BlockSpec((1, rb, d), lambda i: (0, i, 0))
    specdb = pl.BlockSpec((1, rb, d), lambda i: (1, i, 0))
    x_out = pl.pallas_call(
        _fin_body,
        grid=(nb_n,),
        in_specs=[specda, specdb, spec16a, spec16b,
                  pl.BlockSpec((d, d), lambda i: (0, 0)), vspec],
        out_specs=pl.BlockSpec((rb, d), lambda i: (i, 0)),
        out_shape=jax.ShapeDtypeStruct((n, d), jnp.float32),
    )(accn, accn, cntn, cntn,
      W_enc, b_enc.reshape(1, d))

    return (x_out, e)


# speedup vs baseline: 1.6973x; 1.0344x over previous
import functools

import jax
import jax.numpy as jnp
from jax import lax
from jax.experimental import pallas as pl
from jax.experimental.pallas import tpu as pltpu
from jax.experimental.pallas import tpu_sc as plsc

_EPS = 1e-5
_N_EDGES = 20000
_NC = 2
_NS = 16
_NW = _NC * _NS
_K = 80
_ZCH = 16


def _sc_segsum(tables, gidx, sidx, num_segments, k=80, nbuf=4,
               with_count=False, count_segments=0):
    nnz = gidx.shape[0]
    w = tables[0].shape[1]
    assert all(t.shape[1] == w for t in tables)
    nt = len(tables)
    assert w % 16 == 0
    npad = -(-num_segments // (_NS * _ZCH)) * (_NS * _ZCH)
    rpt = npad // _NS
    npad_c = -(-count_segments // (_NS * _ZCH)) * (_NS * _ZCH)
    rpt_c = npad_c // _NS

    nnz_pad = -(-nnz // (_NW * k)) * (_NW * k)
    if nnz_pad != nnz:
        assert not with_count and npad > num_segments
        ext = nnz_pad - nnz
        gidx = jnp.concatenate([gidx, jnp.zeros((ext,), jnp.int32)])
        sidx = jnp.concatenate(
            [sidx, jnp.full((ext,), npad - 1, jnp.int32)])
    q = nnz_pad // _NW
    nbt = q // k

    gidx2 = gidx.reshape(-1, k)
    sidx2 = sidx.reshape(-1, k)


    out_type = [jax.ShapeDtypeStruct((_NC, npad, w), jnp.float32)
                for _ in range(nt)]
    scratch = [
        pltpu.VMEM_SHARED((npad, w), jnp.float32),
        pltpu.VMEM((nbt, k), jnp.int32),
        pltpu.VMEM((nbt, k), jnp.int32),
    ] + [pltpu.VMEM((k, w), jnp.float32) for _ in range(nbuf)] + [
        pltpu.VMEM((_ZCH, w), jnp.float32),
    ] + [pltpu.SemaphoreType.DMA for _ in range(nbuf)]
    if with_count:
        out_type.append(
            jax.ShapeDtypeStruct((_NC, npad_c, 16), jnp.float32))
        scratch += [
            pltpu.VMEM_SHARED((npad_c, 16), jnp.float32),
            pltpu.VMEM((_ZCH, 16), jnp.float32),
            pltpu.VMEM((k, 16), jnp.float32),
        ]

    def body(*args):
        tables_h = args[:nt]
        gidx_h, sidx_h = args[nt], args[nt + 1]
        outs_h = args[nt + 2:2 * nt + 2]
        rest = args[2 * nt + 2:]
        if with_count:
            cnt_h = rest[0]
            rest = rest[1:]
        acc_sh, gi_all, si_all = rest[0], rest[1], rest[2]
        rows = rest[3:3 + nbuf]
        zbuf = rest[3 + nbuf]
        sems = rest[4 + nbuf:4 + 2 * nbuf]
        if with_count:
            cnt_sh, zbuf_c, ones_v = rest[4 + 2 * nbuf:]
        cid = lax.axis_index("c")
        sid = lax.axis_index("s")
        zv = jnp.zeros((16,), jnp.float32)
        wid = cid * _NS + sid

        pltpu.sync_copy(gidx_h.at[pl.ds(wid * nbt, nbt)], gi_all)
        pltpu.sync_copy(sidx_h.at[pl.ds(wid * nbt, nbt)], si_all)

        def fire(tbl, b, j):
            pltpu.async_copy(tbl.at[gi_all.at[b]], rows[j], sems[j])

        def prologue(tbl):
            for j in range(nbuf - 1):
                fire(tbl, j, j)

        prologue(tables_h[0])

        def zero_acc():
            @pl.loop(0, rpt // _ZCH)
            def _(i):
                pltpu.sync_copy(
                    zbuf, acc_sh.at[pl.ds(sid * rpt + i * _ZCH, _ZCH)])

        @pl.loop(0, _ZCH)
        def _(r):
            for c in range(w // 16):
                zbuf[r, pl.ds(c * 16, 16)] = zv

        zero_acc()

        if with_count:
            onev = jnp.where(lax.iota(jnp.int32, 16) == 0, 1.0, 0.0)

            @pl.loop(0, _ZCH)
            def _(r):
                zbuf_c[r, :] = zv

            @pl.loop(0, rpt_c // _ZCH)
            def _(i):
                pltpu.sync_copy(
                    zbuf_c, cnt_sh.at[pl.ds(sid * rpt_c + i * _ZCH, _ZCH)])

            @pl.loop(0, k)
            def _(r):
                ones_v[r, :] = onev

        def wait_gather(j):
            pltpu.make_async_copy(
                tables_h[0].at[gi_all.at[0]], rows[j], sems[j]).wait()

        def retire(b, j):
            wait_gather(j)
            pltpu.sync_copy(rows[j], acc_sh.at[si_all.at[b]], add=True)
            if with_count:
                pltpu.sync_copy(ones_v, cnt_sh.at[gi_all.at[b]], add=True)

        n_main = (nbt - (nbuf - 1)) // nbuf

        for t, tbl in enumerate(tables_h):
            plsc.subcore_barrier()

            @pl.loop(0, n_main)
            def _(i):
                b0 = nbuf * i
                for j in range(nbuf):
                    fire(tbl, b0 + j + nbuf - 1, (j + nbuf - 1) % nbuf)
                    retire(b0 + j, j)

            base = n_main * nbuf
            for b in range(base, nbt):
                m = b + nbuf - 1
                if m < nbt:
                    fire(tbl, m, m % nbuf)
                retire(b, b % nbuf)

            if t + 1 < nt:
                prologue(tables_h[t + 1])

            plsc.subcore_barrier()

            r0 = sid * rpt
            pltpu.sync_copy(acc_sh.at[pl.ds(r0, rpt)],
                            outs_h[t].at[cid, pl.ds(r0, rpt)])
            if t + 1 < nt:
                zero_acc()

        if with_count:
            rc = sid * rpt_c
            pltpu.sync_copy(cnt_sh.at[pl.ds(rc, rpt_c)],
                            cnt_h.at[cid, pl.ds(rc, rpt_c)])

    mesh = plsc.VectorSubcoreMesh(core_axis_name="c", subcore_axis_name="s")
    f = pl.kernel(body, out_type=out_type, mesh=mesh, scratch_types=scratch,
                  compiler_params=pltpu.CompilerParams(
                      use_tc_tiling_on_sc=False),
                  name=f"sc_segsum_w{w}x{nt}{'_cnt' if with_count else ''}")
    return f(*tables, gidx2, sidx2)


def _prep_body(x_ref, x0_ref, x1_ref, y0_ref, y1_ref, s_ref):
    x = x_ref[...]
    mu = jnp.mean(x, axis=1, keepdims=True)
    var = jnp.mean(jnp.square(x - mu), axis=1, keepdims=True)
    inv = lax.rsqrt(var + _EPS)
    y = x * inv
    x0_ref[...] = x[:, :64]
    x1_ref[...] = x[:, 64:]
    y0_ref[...] = y[:, :64]
    y1_ref[...] = y[:, 64:]
    t = mu * inv
    r = x.shape[0]
    col = lax.broadcasted_iota(jnp.int32, (r, 16), 1)
    s_ref[...] = jnp.where(col == 0, t, jnp.where(col == 1, 1.0, 0.0))


def _mid_em_body(x0a, x0b, x1a, x1b, sa, sb, em_ref):
    sx = jnp.concatenate([x0a[0] + x0b[0], x1a[0] + x1b[0]], axis=1)
    s = sa[0] + sb[0]
    deg = s[:, 1:2]
    em_ref[...] = sx / jnp.maximum(deg, 1.0)


def _mid_e_body(y0a, y0b, y1a, y1b, sa, sb, ig, ib, og, ob, e_ref):
    sy = jnp.concatenate([y0a[0] + y0b[0], y1a[0] + y1b[0]], axis=1)
    s = sa[0] + sb[0]
    t_s = s[:, 0:1]
    deg = s[:, 1:2]
    sxn = sy * ig[...] - t_s * ig[...] + deg * ib[...]
    pos = deg > 0.0
    agg = jnp.where(pos, sxn / jnp.where(pos, deg * deg, 1.0), 0.0)
    mu = jnp.mean(agg, axis=1, keepdims=True)
    var = jnp.mean(jnp.square(agg - mu), axis=1, keepdims=True)
    e_ref[...] = (agg - mu) * lax.rsqrt(var + _EPS) * og[...] + ob[...]


def _fin_body(na, nb_, ca, cb, w_ref, b_ref, o_ref):
    nsum = na[0] + nb_[0]
    deg = (ca[0] + cb[0])[:, 0:1]
    nmean = nsum / jnp.maximum(deg, 1.0)
    o_ref[...] = jnp.dot(nmean, w_ref[...],
                         preferred_element_type=jnp.float32) + b_ref[...]


@jax.jit
def kernel(x, edge_index, in_gamma, in_beta, out_gamma, out_beta,
           W_enc, b_enc):
    src = edge_index[0]
    dst = edge_index[1]
    n, d = x.shape
    ne = _N_EDGES
    rb = 1000

    x0, x1, y0, y1, s16 = pl.pallas_call(
        _prep_body,
        grid=(n // rb,),
        in_specs=[pl.BlockSpec((rb, d), lambda i: (i, 0))],
        out_specs=[
            pl.BlockSpec((rb, 64), lambda i: (i, 0)),
            pl.BlockSpec((rb, 64), lambda i: (i, 0)),
            pl.BlockSpec((rb, 64), lambda i: (i, 0)),
            pl.BlockSpec((rb, 64), lambda i: (i, 0)),
            pl.BlockSpec((rb, 16), lambda i: (i, 0)),
        ],
        out_shape=[
            jax.ShapeDtypeStruct((n, 64), jnp.float32),
            jax.ShapeDtypeStruct((n, 64), jnp.float32),
            jax.ShapeDtypeStruct((n, 64), jnp.float32),
            jax.ShapeDtypeStruct((n, 64), jnp.float32),
            jax.ShapeDtypeStruct((n, 16), jnp.float32),
        ],
    )(x)

    accx0, accx1, accy0, accy1 = _sc_segsum([x0, x1, y0, y1], src, dst, ne,
                                            nbuf=5)
    accs, cntn = _sc_segsum([s16], src, dst, ne, nbuf=8, with_count=True,
                            count_segments=n)

    nb_e = ne // rb
    ig = in_gamma.reshape(1, d)
    ib = in_beta.reshape(1, d)
    og = out_gamma.reshape(1, d)
    ob = out_beta.reshape(1, d)
    spec64a = pl.BlockSpec((1, rb, 64), lambda i: (0, i, 0))
    spec64b = pl.BlockSpec((1, rb, 64), lambda i: (1, i, 0))
    spec16a = pl.BlockSpec((1, rb, 16), lambda i: (0, i, 0))
    spec16b = pl.BlockSpec((1, rb, 16), lambda i: (1, i, 0))
    vspec = pl.BlockSpec((1, d), lambda i: (0, 0))
    emean = pl.pallas_call(
        _mid_em_body,
        grid=(nb_e,),
        in_specs=[spec64a, spec64b, spec64a, spec64b, spec16a, spec16b],
        out_specs=pl.BlockSpec((rb, d), lambda i: (i, 0)),
        out_shape=jax.ShapeDtypeStruct((ne, d), jnp.float32),
        compiler_params=pltpu.CompilerParams(
            allow_input_fusion=[True] * 6),
    )(accx0, accx0, accx1, accx1, accs, accs)

    e = pl.pallas_call(
        _mid_e_body,
        grid=(nb_e,),
        in_specs=[spec64a, spec64b, spec64a, spec64b, spec16a, spec16b,
                  vspec, vspec, vspec, vspec],
        out_specs=pl.BlockSpec((rb, d), lambda i: (i, 0)),
        out_shape=jax.ShapeDtypeStruct((ne, d), jnp.float32),
        compiler_params=pltpu.CompilerParams(
            allow_input_fusion=[True] * 6 + [False] * 4),
    )(accy0, accy0, accy1, accy1, accs, accs, ig, ib, og, ob)

    accn, = _sc_segsum([emean], dst, src, n, k=40, nbuf=5)

    nb_n = n // rb
    specda = pl.BlockSpec((1, rb, d), lambda i: (0, i, 0))
    specdb = pl.BlockSpec((1, rb, d), lambda i: (1, i, 0))
    x_out = pl.pallas_call(
        _fin_body,
        grid=(nb_n,),
        in_specs=[specda, specdb, spec16a, spec16b,
                  pl.BlockSpec((d, d), lambda i: (0, 0)), vspec],
        out_specs=pl.BlockSpec((rb, d), lambda i: (i, 0)),
        out_shape=jax.ShapeDtypeStruct((n, d), jnp.float32),
        compiler_params=pltpu.CompilerParams(
            allow_input_fusion=[True] * 4 + [False] * 2),
    )(accn, accn, cntn, cntn,
      W_enc, b_enc.reshape(1, d))

    return (x_out, e)


# speedup vs baseline: 1.7327x; 1.0209x over previous
import functools

import jax
import jax.numpy as jnp
from jax import lax
from jax.experimental import pallas as pl
from jax.experimental.pallas import tpu as pltpu
from jax.experimental.pallas import tpu_sc as plsc

_EPS = 1e-5
_N_EDGES = 20000
_NC = 2
_NS = 16
_NW = _NC * _NS
_K = 80
_ZCH = 16


def _sc_segsum(tables, gidx, sidx, num_segments, k=80, nbuf=4, zch=16,
               with_count=False, count_segments=0):
    nnz = gidx.shape[0]
    w = tables[0].shape[1]
    assert all(t.shape[1] == w for t in tables)
    nt = len(tables)
    assert w % 16 == 0
    npad = -(-num_segments // (_NS * zch)) * (_NS * zch)
    rpt = npad // _NS
    npad_c = -(-count_segments // (_NS * zch)) * (_NS * zch)
    rpt_c = npad_c // _NS

    nnz_pad = -(-nnz // (_NW * k)) * (_NW * k)
    if nnz_pad != nnz:
        assert not with_count and npad > num_segments
        ext = nnz_pad - nnz
        gidx = jnp.concatenate([gidx, jnp.zeros((ext,), jnp.int32)])
        sidx = jnp.concatenate(
            [sidx, jnp.full((ext,), npad - 1, jnp.int32)])
    q = nnz_pad // _NW
    nbt = q // k

    gidx2 = gidx.reshape(-1, k)
    sidx2 = sidx.reshape(-1, k)


    out_type = [jax.ShapeDtypeStruct((_NC, npad, w), jnp.float32)
                for _ in range(nt)]
    scratch = [
        pltpu.VMEM_SHARED((npad, w), jnp.float32),
        pltpu.VMEM((nbt, k), jnp.int32),
        pltpu.VMEM((nbt, k), jnp.int32),
    ] + [pltpu.VMEM((k, w), jnp.float32) for _ in range(nbuf)] + [
        pltpu.VMEM((zch, w), jnp.float32),
    ] + [pltpu.SemaphoreType.DMA for _ in range(nbuf)]
    if with_count:
        out_type.append(
            jax.ShapeDtypeStruct((_NC, npad_c, 16), jnp.float32))
        scratch += [
            pltpu.VMEM_SHARED((npad_c, 16), jnp.float32),
            pltpu.VMEM((zch, 16), jnp.float32),
            pltpu.VMEM((k, 16), jnp.float32),
        ]

    def body(*args):
        tables_h = args[:nt]
        gidx_h, sidx_h = args[nt], args[nt + 1]
        outs_h = args[nt + 2:2 * nt + 2]
        rest = args[2 * nt + 2:]
        if with_count:
            cnt_h = rest[0]
            rest = rest[1:]
        acc_sh, gi_all, si_all = rest[0], rest[1], rest[2]
        rows = rest[3:3 + nbuf]
        zbuf = rest[3 + nbuf]
        sems = rest[4 + nbuf:4 + 2 * nbuf]
        if with_count:
            cnt_sh, zbuf_c, ones_v = rest[4 + 2 * nbuf:]
        cid = lax.axis_index("c")
        sid = lax.axis_index("s")
        zv = jnp.zeros((16,), jnp.float32)
        wid = cid * _NS + sid

        pltpu.sync_copy(gidx_h.at[pl.ds(wid * nbt, nbt)], gi_all)
        pltpu.sync_copy(sidx_h.at[pl.ds(wid * nbt, nbt)], si_all)

        def fire(tbl, b, j):
            pltpu.async_copy(tbl.at[gi_all.at[b]], rows[j], sems[j])

        def prologue(tbl):
            for j in range(nbuf - 1):
                fire(tbl, j, j)

        prologue(tables_h[0])

        def zero_acc():
            @pl.loop(0, rpt // zch)
            def _(i):
                pltpu.sync_copy(
                    zbuf, acc_sh.at[pl.ds(sid * rpt + i * zch, zch)])

        @pl.loop(0, zch)
        def _(r):
            for c in range(w // 16):
                zbuf[r, pl.ds(c * 16, 16)] = zv

        zero_acc()

        if with_count:
            onev = jnp.where(lax.iota(jnp.int32, 16) == 0, 1.0, 0.0)

            @pl.loop(0, zch)
            def _(r):
                zbuf_c[r, :] = zv

            @pl.loop(0, rpt_c // zch)
            def _(i):
                pltpu.sync_copy(
                    zbuf_c, cnt_sh.at[pl.ds(sid * rpt_c + i * zch, zch)])

            @pl.loop(0, k)
            def _(r):
                ones_v[r, :] = onev

        def wait_gather(j):
            pltpu.make_async_copy(
                tables_h[0].at[gi_all.at[0]], rows[j], sems[j]).wait()

        def retire(b, j):
            wait_gather(j)
            pltpu.sync_copy(rows[j], acc_sh.at[si_all.at[b]], add=True)
            if with_count:
                pltpu.sync_copy(ones_v, cnt_sh.at[gi_all.at[b]], add=True)

        n_main = (nbt - (nbuf - 1)) // nbuf

        for t, tbl in enumerate(tables_h):
            plsc.subcore_barrier()

            @pl.loop(0, n_main)
            def _(i):
                b0 = nbuf * i
                for j in range(nbuf):
                    fire(tbl, b0 + j + nbuf - 1, (j + nbuf - 1) % nbuf)
                    retire(b0 + j, j)

            base = n_main * nbuf
            for b in range(base, nbt):
                m = b + nbuf - 1
                if m < nbt:
                    fire(tbl, m, m % nbuf)
                retire(b, b % nbuf)

            if t + 1 < nt:
                prologue(tables_h[t + 1])

            plsc.subcore_barrier()

            r0 = sid * rpt
            pltpu.sync_copy(acc_sh.at[pl.ds(r0, rpt)],
                            outs_h[t].at[cid, pl.ds(r0, rpt)])
            if t + 1 < nt:
                zero_acc()

        if with_count:
            rc = sid * rpt_c
            pltpu.sync_copy(cnt_sh.at[pl.ds(rc, rpt_c)],
                            cnt_h.at[cid, pl.ds(rc, rpt_c)])

    mesh = plsc.VectorSubcoreMesh(core_axis_name="c", subcore_axis_name="s")
    f = pl.kernel(body, out_type=out_type, mesh=mesh, scratch_types=scratch,
                  compiler_params=pltpu.CompilerParams(
                      use_tc_tiling_on_sc=False),
                  name=f"sc_segsum_w{w}x{nt}{'_cnt' if with_count else ''}")
    return f(*tables, gidx2, sidx2)


def _prep_body(x_ref, x0_ref, x1_ref, y0_ref, y1_ref, s_ref):
    x = x_ref[...]
    mu = jnp.mean(x, axis=1, keepdims=True)
    var = jnp.mean(jnp.square(x - mu), axis=1, keepdims=True)
    inv = lax.rsqrt(var + _EPS)
    y = x * inv
    x0_ref[...] = x[:, :64]
    x1_ref[...] = x[:, 64:]
    y0_ref[...] = y[:, :64]
    y1_ref[...] = y[:, 64:]
    t = mu * inv
    r = x.shape[0]
    col = lax.broadcasted_iota(jnp.int32, (r, 16), 1)
    s_ref[...] = jnp.where(col == 0, t, jnp.where(col == 1, 1.0, 0.0))


def _mid_em_body(x0a, x0b, x1a, x1b, sa, sb, em_ref):
    sx = jnp.concatenate([x0a[0] + x0b[0], x1a[0] + x1b[0]], axis=1)
    s = sa[0] + sb[0]
    deg = s[:, 1:2]
    em_ref[...] = sx / jnp.maximum(deg, 1.0)


def _mid_e_body(y0a, y0b, y1a, y1b, sa, sb, ig, ib, og, ob, e_ref):
    sy = jnp.concatenate([y0a[0] + y0b[0], y1a[0] + y1b[0]], axis=1)
    s = sa[0] + sb[0]
    t_s = s[:, 0:1]
    deg = s[:, 1:2]
    sxn = sy * ig[...] - t_s * ig[...] + deg * ib[...]
    pos = deg > 0.0
    agg = jnp.where(pos, sxn / jnp.where(pos, deg * deg, 1.0), 0.0)
    mu = jnp.mean(agg, axis=1, keepdims=True)
    var = jnp.mean(jnp.square(agg - mu), axis=1, keepdims=True)
    e_ref[...] = (agg - mu) * lax.rsqrt(var + _EPS) * og[...] + ob[...]


def _fin_body(na, nb_, ca, cb, w_ref, b_ref, o_ref):
    nsum = na[0] + nb_[0]
    deg = (ca[0] + cb[0])[:, 0:1]
    nmean = nsum / jnp.maximum(deg, 1.0)
    o_ref[...] = jnp.dot(nmean, w_ref[...],
                         preferred_element_type=jnp.float32) + b_ref[...]


@jax.jit
def kernel(x, edge_index, in_gamma, in_beta, out_gamma, out_beta,
           W_enc, b_enc):
    src = edge_index[0]
    dst = edge_index[1]
    n, d = x.shape
    ne = _N_EDGES
    rb = 1000

    x0, x1, y0, y1, s16 = pl.pallas_call(
        _prep_body,
        grid=(n // rb,),
        in_specs=[pl.BlockSpec((rb, d), lambda i: (i, 0))],
        out_specs=[
            pl.BlockSpec((rb, 64), lambda i: (i, 0)),
            pl.BlockSpec((rb, 64), lambda i: (i, 0)),
            pl.BlockSpec((rb, 64), lambda i: (i, 0)),
            pl.BlockSpec((rb, 64), lambda i: (i, 0)),
            pl.BlockSpec((rb, 16), lambda i: (i, 0)),
        ],
        out_shape=[
            jax.ShapeDtypeStruct((n, 64), jnp.float32),
            jax.ShapeDtypeStruct((n, 64), jnp.float32),
            jax.ShapeDtypeStruct((n, 64), jnp.float32),
            jax.ShapeDtypeStruct((n, 64), jnp.float32),
            jax.ShapeDtypeStruct((n, 16), jnp.float32),
        ],
    )(x)

    accx0, accx1, accy0, accy1 = _sc_segsum([x0, x1, y0, y1], src, dst, ne,
                                            nbuf=4, zch=79)
    accs, cntn = _sc_segsum([s16], src, dst, ne, nbuf=8, zch=79,
                            with_count=True, count_segments=n)

    nb_e = ne // rb
    ig = in_gamma.reshape(1, d)
    ib = in_beta.reshape(1, d)
    og = out_gamma.reshape(1, d)
    ob = out_beta.reshape(1, d)
    spec64a = pl.BlockSpec((1, rb, 64), lambda i: (0, i, 0))
    spec64b = pl.BlockSpec((1, rb, 64), lambda i: (1, i, 0))
    spec16a = pl.BlockSpec((1, rb, 16), lambda i: (0, i, 0))
    spec16b = pl.BlockSpec((1, rb, 16), lambda i: (1, i, 0))
    vspec = pl.BlockSpec((1, d), lambda i: (0, 0))
    emean = pl.pallas_call(
        _mid_em_body,
        grid=(nb_e,),
        in_specs=[spec64a, spec64b, spec64a, spec64b, spec16a, spec16b],
        out_specs=pl.BlockSpec((rb, d), lambda i: (i, 0)),
        out_shape=jax.ShapeDtypeStruct((ne, d), jnp.float32),
        compiler_params=pltpu.CompilerParams(
            allow_input_fusion=[True] * 6),
    )(accx0, accx0, accx1, accx1, accs, accs)

    e = pl.pallas_call(
        _mid_e_body,
        grid=(nb_e,),
        in_specs=[spec64a, spec64b, spec64a, spec64b, spec16a, spec16b,
                  vspec, vspec, vspec, vspec],
        out_specs=pl.BlockSpec((rb, d), lambda i: (i, 0)),
        out_shape=jax.ShapeDtypeStruct((ne, d), jnp.float32),
        compiler_params=pltpu.CompilerParams(
            allow_input_fusion=[True] * 6 + [False] * 4),
    )(accy0, accy0, accy1, accy1, accs, accs, ig, ib, og, ob)

    accn, = _sc_segsum([emean], dst, src, n, k=40, nbuf=5)

    nb_n = n // rb
    specda = pl.BlockSpec((1, rb, d), lambda i: (0, i, 0))
    specdb = pl.BlockSpec((1, rb, d), lambda i: (1, i, 0))
    x_out = pl.pallas_call(
        _fin_body,
        grid=(nb_n,),
        in_specs=[specda, specdb, spec16a, spec16b,
                  pl.BlockSpec((d, d), lambda i: (0, 0)), vspec],
        out_specs=pl.BlockSpec((rb, d), lambda i: (i, 0)),
        out_shape=jax.ShapeDtypeStruct((n, d), jnp.float32),
        compiler_params=pltpu.CompilerParams(
            allow_input_fusion=[True] * 4 + [False] * 2),
    )(accn, accn, cntn, cntn,
      W_enc, b_enc.reshape(1, d))

    return (x_out, e)
